# baseline (speedup 1.0000x reference)
import functools

import jax
import jax.numpy as jnp
from jax.experimental import pallas as pl

_MASK_RATIO = 0.5
_EPS = 1e-6


def _entropy_body(x_ref, ent_ref):
    blk = x_ref[...]
    m = jnp.max(blk, axis=1, keepdims=True)
    e = jnp.exp(blk - m)
    s = jnp.sum(e, axis=1, keepdims=True)
    q = e / s + _EPS
    ent_ref[0, 0, :] = -jnp.sum(q * jnp.log(q), axis=1)


def _mask_body(ent_ref, mask_ref, *, k):
    ent = ent_ref[...]
    b, c = ent.shape
    ent_i = ent[:, :, None]
    ent_j = ent[:, None, :]
    lt = (ent_j < ent_i).astype(jnp.float32)
    eq = ent_j == ent_i
    jj = jax.lax.broadcasted_iota(jnp.int32, (b, c, c), 2)
    ii = jax.lax.broadcasted_iota(jnp.int32, (b, c, c), 1)
    tie = (eq & (jj < ii)).astype(jnp.float32)
    rank = jnp.sum(lt + tie, axis=2)
    mask_ref[...] = (rank < k).astype(jnp.float32)


def _mul_body(mask_ref, x_ref, o_ref):
    m = mask_ref[0, 0, :]
    o_ref[...] = x_ref[...] * m[:, None]


def kernel(x):
    B, C, H, W = x.shape
    HW = H * W
    k = int(C * (1 - _MASK_RATIO))
    CB = 8
    N = (B * C) // CB
    x2 = x.reshape(B * C, HW)

    ent = pl.pallas_call(
        _entropy_body,
        grid=(N,),
        in_specs=[pl.BlockSpec((CB, HW), lambda i: (i, 0))],
        out_specs=pl.BlockSpec((1, 1, CB), lambda i: (i, 0, 0)),
        out_shape=jax.ShapeDtypeStruct((N, 1, CB), jnp.float32),
    )(x2)

    mask = pl.pallas_call(
        functools.partial(_mask_body, k=k),
        in_specs=[pl.BlockSpec((B, C), lambda: (0, 0))],
        out_specs=pl.BlockSpec((B, C), lambda: (0, 0)),
        out_shape=jax.ShapeDtypeStruct((B, C), jnp.float32),
    )(ent.reshape(B, C))

    out = pl.pallas_call(
        _mul_body,
        grid=(N,),
        in_specs=[
            pl.BlockSpec((1, 1, CB), lambda i: (i, 0, 0)),
            pl.BlockSpec((CB, HW), lambda i: (i, 0)),
        ],
        out_specs=pl.BlockSpec((CB, HW), lambda i: (i, 0)),
        out_shape=jax.ShapeDtypeStruct((B * C, HW), jnp.float32),
    )(mask.reshape(N, 1, CB), x2)

    return out.reshape(B, C, H, W)


# speedup vs baseline: 1.3986x; 1.3986x over previous
import functools

import jax
import jax.numpy as jnp
from jax.experimental import pallas as pl

_MASK_RATIO = 0.5
_EPS = 1e-6


def _entropy_body(x_ref, ent_ref):
    blk4 = x_ref[...]
    cb = blk4.shape[1]
    blk = blk4.reshape(cb, blk4.shape[2] * blk4.shape[3])
    m = jnp.max(blk, axis=1, keepdims=True)
    e = jnp.exp(blk - m)
    s = jnp.sum(e, axis=1, keepdims=True)
    q = e / s + _EPS
    ent_ref[0, 0, :] = -jnp.sum(q * jnp.log(q), axis=1)


def _mask_body(ent_ref, mask_ref, *, k):
    ent = ent_ref[...]
    b, c = ent.shape
    ent_i = ent[:, :, None]
    ent_j = ent[:, None, :]
    lt = (ent_j < ent_i).astype(jnp.float32)
    eq = ent_j == ent_i
    jj = jax.lax.broadcasted_iota(jnp.int32, (b, c, c), 2)
    ii = jax.lax.broadcasted_iota(jnp.int32, (b, c, c), 1)
    tie = (eq & (jj < ii)).astype(jnp.float32)
    rank = jnp.sum(lt + tie, axis=2)
    mask_ref[...] = (rank < k).astype(jnp.float32)


def _mul_body(mask_ref, x_ref, o_ref):
    m = mask_ref[0, 0, :]
    o_ref[...] = x_ref[...] * m[None, :, None, None]


def kernel(x):
    B, C, H, W = x.shape
    k = int(C * (1 - _MASK_RATIO))
    CB = 8
    N = (B * C) // CB
    CBC = C // CB

    ent = pl.pallas_call(
        _entropy_body,
        grid=(N,),
        in_specs=[pl.BlockSpec((1, CB, H, W), lambda i: (i // CBC, i % CBC, 0, 0))],
        out_specs=pl.BlockSpec((1, 1, CB), lambda i: (i, 0, 0)),
        out_shape=jax.ShapeDtypeStruct((N, 1, CB), jnp.float32),
    )(x)

    mask = pl.pallas_call(
        functools.partial(_mask_body, k=k),
        in_specs=[pl.BlockSpec((B, C), lambda: (0, 0))],
        out_specs=pl.BlockSpec((B, C), lambda: (0, 0)),
        out_shape=jax.ShapeDtypeStruct((B, C), jnp.float32),
    )(ent.reshape(B, C))

    out = pl.pallas_call(
        _mul_body,
        grid=(N,),
        in_specs=[
            pl.BlockSpec((1, 1, CB), lambda i: (i, 0, 0)),
            pl.BlockSpec((1, CB, H, W), lambda i: (i // CBC, i % CBC, 0, 0)),
        ],
        out_specs=pl.BlockSpec((1, CB, H, W), lambda i: (i // CBC, i % CBC, 0, 0)),
        out_shape=jax.ShapeDtypeStruct((B, C, H, W), jnp.float32),
    )(mask.reshape(N, 1, CB), x)

    return out


# speedup vs baseline: 3.4197x; 2.4452x over previous
import functools

import jax
import jax.numpy as jnp
from jax.experimental import pallas as pl
from jax.experimental.pallas import tpu as pltpu

_MASK_RATIO = 0.5
_EPS = 1e-6


def _max_body(y_ref, m_ref):
    h = pl.program_id(1)
    blk = y_ref[...]
    p = jnp.max(blk, axis=(0, 1, 2))
    prev = jnp.where(h == 0, jnp.full_like(p, -jnp.inf), m_ref[0, 0, :])
    m_ref[0, 0, :] = jnp.maximum(prev, p)


def _sumexp_body(y_ref, m_ref, s_ref, acc_ref, *, nh, wg):
    h = pl.program_id(1)
    blk = y_ref[...]
    m = m_ref[0, 0, :]
    e = jnp.exp(blk - m[None, None, None, :])
    hb, w, c = e.shape[1], e.shape[2], e.shape[3]
    part = jnp.sum(e.reshape(hb * (w // wg), wg, c), axis=0)
    acc_ref[...] = jnp.where(h == 0, part, acc_ref[...] + part)

    @pl.when(h == nh - 1)
    def _():
        s_ref[0, 0, :] = jnp.sum(acc_ref[...], axis=0)


def _ent_body(y_ref, m_ref, s_ref, e_ref, acc_ref, *, nh, wg):
    h = pl.program_id(1)
    blk = y_ref[...]
    m = m_ref[0, 0, :]
    s = s_ref[0, 0, :]
    e = jnp.exp(blk - m[None, None, None, :])
    q = e / s[None, None, None, :] + _EPS
    t = q * jnp.log(q)
    hb, w, c = t.shape[1], t.shape[2], t.shape[3]
    part = jnp.sum(t.reshape(hb * (w // wg), wg, c), axis=0)
    acc_ref[...] = jnp.where(h == 0, part, acc_ref[...] + part)

    @pl.when(h == nh - 1)
    def _():
        e_ref[0, 0, :] = -jnp.sum(acc_ref[...], axis=0)


def _mask_body(ent_ref, mask_ref, *, k):
    ent = ent_ref[:, 0, :]
    b, c = ent.shape
    ent_i = ent[:, :, None]
    ent_j = ent[:, None, :]
    lt = (ent_j < ent_i).astype(jnp.float32)
    eq = ent_j == ent_i
    jj = jax.lax.broadcasted_iota(jnp.int32, (b, c, c), 2)
    ii = jax.lax.broadcasted_iota(jnp.int32, (b, c, c), 1)
    tie = (eq & (jj < ii)).astype(jnp.float32)
    rank = jnp.sum(lt + tie, axis=2)
    mask_ref[:, 0, :] = (rank < k).astype(jnp.float32)


def _mul_body(mask_ref, y_ref, o_ref):
    mk = mask_ref[0, 0, :]
    o_ref[...] = y_ref[...] * mk[None, None, None, :]


def kernel(x):
    B, C, H, W = x.shape
    k = int(C * (1 - _MASK_RATIO))
    y = jnp.transpose(x, (0, 2, 3, 1))
    HB = 28
    NH = H // HB
    WG = 8

    ysp = pl.BlockSpec((1, HB, W, C), lambda b, h: (b, h, 0, 0))
    csp = pl.BlockSpec((1, 1, C), lambda b, h: (b, 0, 0))

    m = pl.pallas_call(
        _max_body,
        grid=(B, NH),
        in_specs=[ysp],
        out_specs=csp,
        out_shape=jax.ShapeDtypeStruct((B, 1, C), jnp.float32),
    )(y)

    s = pl.pallas_call(
        functools.partial(_sumexp_body, nh=NH, wg=WG),
        grid=(B, NH),
        in_specs=[ysp, csp],
        out_specs=csp,
        out_shape=jax.ShapeDtypeStruct((B, 1, C), jnp.float32),
        scratch_shapes=[pltpu.VMEM((WG, C), jnp.float32)],
    )(y, m)

    ent = pl.pallas_call(
        functools.partial(_ent_body, nh=NH, wg=WG),
        grid=(B, NH),
        in_specs=[ysp, csp, csp],
        out_specs=csp,
        out_shape=jax.ShapeDtypeStruct((B, 1, C), jnp.float32),
        scratch_shapes=[pltpu.VMEM((WG, C), jnp.float32)],
    )(y, m, s)

    mask = pl.pallas_call(
        functools.partial(_mask_body, k=k),
        in_specs=[pl.BlockSpec((B, 1, C), lambda: (0, 0, 0))],
        out_specs=pl.BlockSpec((B, 1, C), lambda: (0, 0, 0)),
        out_shape=jax.ShapeDtypeStruct((B, 1, C), jnp.float32),
    )(ent)

    z = pl.pallas_call(
        _mul_body,
        grid=(B, NH),
        in_specs=[csp, ysp],
        out_specs=ysp,
        out_shape=jax.ShapeDtypeStruct((B, H, W, C), jnp.float32),
    )(mask, y)

    return jnp.transpose(z, (0, 3, 1, 2))
